# baseline (speedup 1.0000x reference)
import jax
import jax.numpy as jnp
from jax import lax
from jax.experimental import pallas as pl
from jax.experimental.pallas import tpu as pltpu

_N = 55
_P = 56
_H = 64
_H2 = 128
_L = 5
_PP = _P * _P
_CR = 3.0


def _pair_diffs(coord_c, coord_r):
    out = []
    for k in range(3):
        col = coord_c[:, k:k + 1]
        row = coord_r[k:k + 1, :]
        out.append(col[:, None, :] - row[:, :, None])
    return out


def _radial(d3):
    return d3[0] * d3[0] + d3[1] * d3[1] + d3[2] * d3[2]


def _fwd_kernel(t_ref, d_ref, x_ref, xt_ref, embW_ref, embb_ref,
                eW1a_ref, eW1b_ref, ewr_ref, ewe_ref, eb1_ref,
                eW2_ref, eb2_ref,
                nW1a_ref, nW1b_ref, nb1_ref, nW2_ref, nb2_ref,
                cW1_ref, cb1_ref, cW2r_ref,
                aWr_ref, ab_ref,
                out_ref):
    f32 = jnp.float32
    silu = jax.nn.silu

    bi = lax.broadcasted_iota(jnp.int32, (_P, _P, 1), 0)
    bj = lax.broadcasted_iota(jnp.int32, (_P, _P, 1), 1)
    mask_agg = ((bi != bj) & (bj < _N)).astype(f32).reshape(_PP, 1)
    mask_j3 = (bj < _N).astype(f32)

    zc = jnp.zeros((_P - _N, 3), f32)
    zr = jnp.zeros((3, _P - _N), f32)
    x0c = [jnp.concatenate([x_ref[0, s], zc], axis=0) for s in (0, 1)]
    x0r = [jnp.concatenate([xt_ref[0, s], zr], axis=1) for s in (0, 1)]

    h0 = (t_ref[0] * embW_ref[0:1, :] + d_ref[0] * embW_ref[1:2, :]
          + embb_ref[...])
    h = jnp.broadcast_to(
        jnp.concatenate([h0[0:1, :], h0[1:2, :]], axis=1), (_P, _H2))

    d3 = [_pair_diffs(x0c[s], x0r[s]) for s in (0, 1)]
    rad3 = [_radial(d3[s]) for s in (0, 1)]
    ea3 = rad3

    coord_c = list(x0c)
    coord_r = list(x0r)
    for l in range(_L):
        if l:
            d3 = [_pair_diffs(coord_c[s], coord_r[s]) for s in (0, 1)]
            rad3 = [_radial(d3[s]) for s in (0, 1)]
        inv3 = [1.0 / (jnp.sqrt(rad3[s]) + 1.0) for s in (0, 1)]

        preI = jnp.dot(h, eW1a_ref[l], preferred_element_type=f32) \
            + eb1_ref[l]
        preJ = jnp.dot(h, eW1b_ref[l], preferred_element_type=f32)
        rw = jnp.concatenate(
            [rad3[0] * ewr_ref[l][None] + ea3[0] * ewe_ref[l][None],
             rad3[1] * ewr_ref[l][None] + ea3[1] * ewe_ref[l][None]],
            axis=2)
        m1 = silu(preI[:, None, :] + preJ[None, :, :] + rw).reshape(_PP, _H2)
        m2 = silu(jnp.dot(m1, eW2_ref[l], preferred_element_type=f32)
                  + eb2_ref[l])
        av = m2 * aWr_ref[l]
        att = [jax.nn.sigmoid(
            jnp.sum(av[:, s * _H:(s + 1) * _H], axis=1, keepdims=True)
            + ab_ref[l]) for s in (0, 1)]
        m = m2 * jnp.concatenate(
            [jnp.broadcast_to(att[0], (_PP, _H)),
             jnp.broadcast_to(att[1], (_PP, _H))], axis=1)
        cm = silu(jnp.dot(m, cW1_ref[l], preferred_element_type=f32)
                  + cb1_ref[l])
        cv = cm * cW2r_ref[l]
        for s in (0, 1):
            cms = jnp.tanh(
                jnp.sum(cv[:, s * _H:(s + 1) * _H], axis=1, keepdims=True))
            ts3 = cms.reshape(_P, _P, 1) * inv3[s] * (mask_j3 * _CR)
            delta_c = jnp.concatenate(
                [jnp.sum(d3[s][k] * ts3, axis=1) for k in range(3)], axis=1)
            coord_c[s] = coord_c[s] + delta_c
            coord_r[s] = coord_r[s] + delta_c.T

        agg = jnp.sum((m * mask_agg).reshape(_P, _P, _H2), axis=1)
        hn = silu(jnp.dot(h, nW1a_ref[l], preferred_element_type=f32)
                  + jnp.dot(agg, nW1b_ref[l], preferred_element_type=f32)
                  + nb1_ref[l])
        h = h + jnp.dot(hn, nW2_ref[l], preferred_element_type=f32) \
            + nb2_ref[l]

    for s in (0, 1):
        vel = (coord_c[s] - x0c[s])[:_N, :]
        vel = vel - jnp.sum(vel, axis=0, keepdims=True) * (1.0 / _N)
        out_ref[0, s] = vel


def _blockdiag(w):
    z = jnp.zeros_like(w)
    return jnp.concatenate(
        [jnp.concatenate([w, z], axis=2),
         jnp.concatenate([z, w], axis=2)], axis=1)


def _lanes2(b):
    return jnp.concatenate([b, b], axis=2)


def kernel(t, x, d_base, emb_W, emb_b, edge_W1, edge_b1, edge_W2, edge_b2,
           node_W1, node_b1, node_W2, node_b2, coord_W1, coord_b1, coord_W2,
           att_W, att_b):
    B = t.shape[0]
    G = B // 2
    x4 = x.reshape(G, 2, _N, 3)
    xt = jnp.swapaxes(x4, 2, 3)
    t3 = t.reshape(G, 2, 1)
    db3 = d_base.reshape(G, 2, 1)

    eW1a = _blockdiag(edge_W1[:, :_H, :])
    eW1b = _blockdiag(edge_W1[:, _H:2 * _H, :])
    ewr = edge_W1[:, 2 * _H:2 * _H + 1, :]
    ewe = edge_W1[:, 2 * _H + 1:, :]
    operands = (
        t3, db3, x4, xt, emb_W, emb_b.reshape(1, _H),
        eW1a, eW1b, ewr, ewe, _lanes2(edge_b1[:, None, :]),
        _blockdiag(edge_W2), _lanes2(edge_b2[:, None, :]),
        _blockdiag(node_W1[:, :_H, :]), _blockdiag(node_W1[:, _H:, :]),
        _lanes2(node_b1[:, None, :]), _blockdiag(node_W2),
        _lanes2(node_b2[:, None, :]),
        _blockdiag(coord_W1), _lanes2(coord_b1[:, None, :]),
        _lanes2(jnp.swapaxes(coord_W2, 1, 2)),
        _lanes2(jnp.swapaxes(att_W, 1, 2)), att_b[:, :, None],
    )

    def batched(a):
        bs = (1,) + a.shape[1:]
        return pl.BlockSpec(bs, lambda b: (b,) + (0,) * (a.ndim - 1))

    def full(a):
        return pl.BlockSpec(a.shape, lambda b: (0,) * a.ndim)

    in_specs = [batched(o) for o in operands[:4]] + \
               [full(o) for o in operands[4:]]

    out = pl.pallas_call(
        _fwd_kernel,
        grid=(G,),
        in_specs=in_specs,
        out_specs=pl.BlockSpec((1, 2, _N, 3), lambda b: (b, 0, 0, 0)),
        out_shape=jax.ShapeDtypeStruct((G, 2, _N, 3), jnp.float32),
        compiler_params=pltpu.CompilerParams(
            dimension_semantics=("parallel",)),
    )(*operands)
    return out.reshape(B, _N * 3)


# speedup vs baseline: 1.5986x; 1.5986x over previous
import jax
import jax.numpy as jnp
from jax import lax
from jax.experimental import pallas as pl
from jax.experimental.pallas import tpu as pltpu

_N = 55
_P = 56
_H = 64
_L = 5
_PP = _P * _P
_CR = 3.0


def _fwd_kernel(t_ref, d_ref, x_ref, xt_ref,
                scisrj_ref, sci_ref, srj_ref, selT_ref, agg_ref,
                embW_ref, embb_ref,
                eW1a_ref, eW1b_ref, ewr_ref, ewe_ref, eb1_ref,
                eW2_ref, eb2_ref,
                nW1a_ref, nW1b_ref, nb1_ref, nW2_ref, nb2_ref,
                cW1_ref, cb1_ref, cW2_ref,
                aW_ref, ab_ref,
                out_ref):
    f32 = jnp.float32
    silu = jax.nn.silu

    def mm(a, b):
        return jnp.dot(a, b, preferred_element_type=f32)

    jsub = lax.broadcasted_iota(jnp.int32, (_P, _P), 0)
    maskt_cr = (jsub < _N).astype(f32) * _CR

    x0c = jnp.concatenate(
        [x_ref[0], jnp.zeros((_P - _N, 3), f32)], axis=0)
    x0r = jnp.concatenate(
        [xt_ref[0], jnp.zeros((3, _P - _N), f32)], axis=1)

    h0 = (t_ref[0] * embW_ref[0:1, :] + d_ref[0] * embW_ref[1:2, :]
          + embb_ref[...])
    h = jnp.broadcast_to(h0, (_P, _H))

    def diffs(cc, cr):
        return [cr[k:k + 1, :] - cc[:, k:k + 1] for k in range(3)]

    e0 = diffs(x0c, x0r)
    radT0 = e0[0] * e0[0] + e0[1] * e0[1] + e0[2] * e0[2]
    a_ea = mm(srj_ref[...], radT0) * sci_ref[...]

    coord_c, coord_r = x0c, x0r
    e2, radT = e0, radT0
    for l in range(_L):
        if l:
            e2 = diffs(coord_c, coord_r)
            radT = e2[0] * e2[0] + e2[1] * e2[1] + e2[2] * e2[2]
        invT = 1.0 / (jnp.sqrt(radT) + 1.0)

        preI = mm(h, eW1a_ref[l]) + eb1_ref[l]
        preJ = mm(h, eW1b_ref[l])
        a_rad = mm(srj_ref[...], radT) * sci_ref[...]
        m1 = silu(mm(scisrj_ref[...],
                     jnp.concatenate([preI, preJ], axis=0))
                  + mm(a_rad, jnp.broadcast_to(ewr_ref[l], (_P, _H)))
                  + mm(a_ea, jnp.broadcast_to(ewe_ref[l], (_P, _H))))
        m2 = silu(mm(m1, eW2_ref[l]) + eb2_ref[l])
        att = jax.nn.sigmoid(mm(m2, aW_ref[l]) + ab_ref[l])
        m = m2 * att
        cm = silu(mm(m, cW1_ref[l]) + cb1_ref[l])
        cms = mm(cm, cW2_ref[l])
        gate = jnp.tanh(mm(selT_ref[...], cms * sci_ref[...]))
        ts = gate * invT * maskt_cr
        drow = jnp.concatenate(
            [jnp.sum(e2[k] * ts, axis=0, keepdims=True) for k in range(3)],
            axis=0)
        coord_r = coord_r + drow
        coord_c = coord_c + drow.T

        agg = mm(agg_ref[...], m)
        hn = silu(mm(h, nW1a_ref[l]) + mm(agg, nW1b_ref[l])
                  + nb1_ref[l])
        h = h + mm(hn, nW2_ref[l]) + nb2_ref[l]

    vel = (coord_c - x0c)[:_N, :]
    vel = vel - jnp.sum(vel, axis=0, keepdims=True) * (1.0 / _N)
    out_ref[0] = vel


def kernel(t, x, d_base, emb_W, emb_b, edge_W1, edge_b1, edge_W2, edge_b2,
           node_W1, node_b1, node_W2, node_b2, coord_W1, coord_b1, coord_W2,
           att_W, att_b):
    B = t.shape[0]
    x3 = x.reshape(B, _N, 3)
    xt = jnp.swapaxes(x3, 1, 2)
    t3 = t.reshape(B, 1, 1)
    db3 = d_base.reshape(B, 1, 1)

    pidx = jnp.arange(_PP, dtype=jnp.int32)
    pi, pj = pidx // _P, pidx % _P
    k56 = jnp.arange(_P, dtype=jnp.int32)
    sci = (pi[:, None] == k56[None, :]).astype(jnp.float32)
    srj = (pj[:, None] == k56[None, :]).astype(jnp.float32)
    scisrj = jnp.concatenate([sci, srj], axis=1)
    selT = srj.T
    agg_sel = ((pi[None, :] == k56[:, None])
               & (pj[None, :] != k56[:, None])
               & (pj[None, :] < _N)).astype(jnp.float32)

    eW1a = edge_W1[:, :_H, :]
    eW1b = edge_W1[:, _H:2 * _H, :]
    ewr = edge_W1[:, 2 * _H:2 * _H + 1, :]
    ewe = edge_W1[:, 2 * _H + 1:, :]
    nW1a = node_W1[:, :_H, :]
    nW1b = node_W1[:, _H:, :]
    operands = (
        t3, db3, x3, xt,
        scisrj, sci, srj, selT, agg_sel,
        emb_W, emb_b.reshape(1, _H),
        eW1a, eW1b, ewr, ewe, edge_b1[:, None, :],
        edge_W2, edge_b2[:, None, :],
        nW1a, nW1b, node_b1[:, None, :], node_W2, node_b2[:, None, :],
        coord_W1, coord_b1[:, None, :], coord_W2,
        att_W, att_b[:, :, None],
    )

    def batched(a):
        bs = (1,) + a.shape[1:]
        return pl.BlockSpec(bs, lambda b: (b,) + (0,) * (a.ndim - 1))

    def full(a):
        return pl.BlockSpec(a.shape, lambda b: (0,) * a.ndim)

    in_specs = [batched(o) for o in operands[:4]] + \
               [full(o) for o in operands[4:]]

    out = pl.pallas_call(
        _fwd_kernel,
        grid=(B,),
        in_specs=in_specs,
        out_specs=pl.BlockSpec((1, _N, 3), lambda b: (b, 0, 0)),
        out_shape=jax.ShapeDtypeStruct((B, _N, 3), jnp.float32),
        compiler_params=pltpu.CompilerParams(
            dimension_semantics=("parallel",)),
    )(*operands)
    return out.reshape(B, _N * 3)


# speedup vs baseline: 2.0787x; 1.3003x over previous
import jax
import jax.numpy as jnp
from jax import lax
from jax.experimental import pallas as pl
from jax.experimental.pallas import tpu as pltpu

_N = 55
_P = 56
_H = 64
_H2 = 128
_L = 5
_PP = _P * _P
_CR = 3.0


def _fwd_kernel(t_ref, d_ref, x_ref, xt_ref,
                scisrj_ref, sci_ref, srj_ref, scisci_ref, selT_ref, agg_ref,
                e2b_ref, e56_ref,
                embW_ref, embb_ref,
                eW1a_ref, eW1b_ref, ewrL_ref, ewrR_ref, eweL_ref, eweR_ref,
                eb1_ref, eW2_ref, eb2_ref,
                nW1a_ref, nW1b_ref, nb1_ref, nW2_ref, nb2_ref,
                cW1_ref, cb1_ref, cW2_ref,
                aW_ref, ab_ref,
                out_ref):
    f32 = jnp.float32
    silu = jax.nn.silu

    def mm(a, b):
        return jnp.dot(a, b, preferred_element_type=f32)

    jsub = lax.broadcasted_iota(jnp.int32, (_P, _P), 0)
    maskt_cr = (jsub < _N).astype(f32) * _CR

    zc = jnp.zeros((_P - _N, 3), f32)
    zr = jnp.zeros((3, _P - _N), f32)
    x0c = [jnp.concatenate([x_ref[0, s], zc], axis=0) for s in (0, 1)]
    x0r = [jnp.concatenate([xt_ref[0, s], zr], axis=1) for s in (0, 1)]

    h0 = (t_ref[0] * embW_ref[0:1, :] + d_ref[0] * embW_ref[1:2, :]
          + embb_ref[...])
    h = jnp.broadcast_to(
        jnp.concatenate([h0[0:1, :], h0[1:2, :]], axis=1), (_P, _H2))

    def diffs(cc, cr):
        return [cr[k:k + 1, :] - cc[:, k:k + 1] for k in range(3)]

    def radial(e):
        return e[0] * e[0] + e[1] * e[1] + e[2] * e[2]

    e0 = [diffs(x0c[s], x0r[s]) for s in (0, 1)]
    radT0 = [radial(e0[s]) for s in (0, 1)]
    a_ea = [mm(srj_ref[...], radT0[s]) * sci_ref[...] for s in (0, 1)]

    coord_c = list(x0c)
    coord_r = list(x0r)
    e2, radT = e0, radT0
    for l in range(_L):
        if l:
            e2 = [diffs(coord_c[s], coord_r[s]) for s in (0, 1)]
            radT = [radial(e2[s]) for s in (0, 1)]
        invT = [1.0 / (jnp.sqrt(radT[s]) + 1.0) for s in (0, 1)]

        preI = mm(h, eW1a_ref[l]) + eb1_ref[l]
        preJ = mm(h, eW1b_ref[l])
        a_rad = [mm(srj_ref[...], radT[s]) * sci_ref[...] for s in (0, 1)]
        m1 = silu(mm(scisrj_ref[...],
                     jnp.concatenate([preI, preJ], axis=0))
                  + mm(a_rad[0], ewrL_ref[l]) + mm(a_rad[1], ewrR_ref[l])
                  + mm(a_ea[0], eweL_ref[l]) + mm(a_ea[1], eweR_ref[l]))
        m2 = silu(mm(m1, eW2_ref[l]) + eb2_ref[l])
        att = jax.nn.sigmoid(mm(m2, aW_ref[l]) + ab_ref[l])
        m = m2 * mm(att, e2b_ref[...])
        cm = silu(mm(m, cW1_ref[l]) + cb1_ref[l])
        cms = mm(cm, cW2_ref[l])
        gates = jnp.tanh(
            mm(selT_ref[...],
               mm(cms, e56_ref[...]) * scisci_ref[...]))
        agg = mm(agg_ref[...], m)
        for s in (0, 1):
            ts = gates[:, s * _P:(s + 1) * _P] * invT[s] * maskt_cr
            drow = jnp.concatenate(
                [jnp.sum(e2[s][k] * ts, axis=0, keepdims=True)
                 for k in range(3)], axis=0)
            coord_r[s] = coord_r[s] + drow
            coord_c[s] = coord_c[s] + drow.T

        hn = silu(mm(h, nW1a_ref[l]) + mm(agg, nW1b_ref[l])
                  + nb1_ref[l])
        h = h + mm(hn, nW2_ref[l]) + nb2_ref[l]

    for s in (0, 1):
        vel = (coord_c[s] - x0c[s])[:_N, :]
        vel = vel - jnp.sum(vel, axis=0, keepdims=True) * (1.0 / _N)
        out_ref[0, s] = vel


def _blockdiag(w):
    z = jnp.zeros_like(w)
    return jnp.concatenate(
        [jnp.concatenate([w, z], axis=2),
         jnp.concatenate([z, w], axis=2)], axis=1)


def _lanes2(b):
    return jnp.concatenate([b, b], axis=2)


def kernel(t, x, d_base, emb_W, emb_b, edge_W1, edge_b1, edge_W2, edge_b2,
           node_W1, node_b1, node_W2, node_b2, coord_W1, coord_b1, coord_W2,
           att_W, att_b):
    B = t.shape[0]
    G = B // 2
    x4 = x.reshape(G, 2, _N, 3)
    xt = jnp.swapaxes(x4, 2, 3)
    t3 = t.reshape(G, 2, 1)
    db3 = d_base.reshape(G, 2, 1)

    pidx = jnp.arange(_PP, dtype=jnp.int32)
    pi, pj = pidx // _P, pidx % _P
    k56 = jnp.arange(_P, dtype=jnp.int32)
    sci = (pi[:, None] == k56[None, :]).astype(jnp.float32)
    srj = (pj[:, None] == k56[None, :]).astype(jnp.float32)
    scisrj = jnp.concatenate([sci, srj], axis=1)
    scisci = jnp.concatenate([sci, sci], axis=1)
    selT = srj.T
    agg_sel = ((pi[None, :] == k56[:, None])
               & (pj[None, :] != k56[:, None])
               & (pj[None, :] < _N)).astype(jnp.float32)
    lane = jnp.arange(_H2)
    e2b = jnp.stack([(lane < _H).astype(jnp.float32),
                     (lane >= _H).astype(jnp.float32)], axis=0)
    lane2 = jnp.arange(2 * _P)
    e56 = jnp.stack([(lane2 < _P).astype(jnp.float32),
                     (lane2 >= _P).astype(jnp.float32)], axis=0)

    def half_rows(w, right):
        z = jnp.zeros_like(w)
        row = jnp.concatenate([z, w] if right else [w, z], axis=2)
        return jnp.broadcast_to(row, (_L, _P, _H2))

    ewr = edge_W1[:, 2 * _H:2 * _H + 1, :]
    ewe = edge_W1[:, 2 * _H + 1:, :]
    operands = (
        t3, db3, x4, xt,
        scisrj, sci, srj, scisci, selT, agg_sel, e2b, e56,
        emb_W, emb_b.reshape(1, _H),
        _blockdiag(edge_W1[:, :_H, :]), _blockdiag(edge_W1[:, _H:2 * _H, :]),
        half_rows(ewr, False), half_rows(ewr, True),
        half_rows(ewe, False), half_rows(ewe, True),
        _lanes2(edge_b1[:, None, :]),
        _blockdiag(edge_W2), _lanes2(edge_b2[:, None, :]),
        _blockdiag(node_W1[:, :_H, :]), _blockdiag(node_W1[:, _H:, :]),
        _lanes2(node_b1[:, None, :]), _blockdiag(node_W2),
        _lanes2(node_b2[:, None, :]),
        _blockdiag(coord_W1), _lanes2(coord_b1[:, None, :]),
        _blockdiag(coord_W2),
        _blockdiag(att_W), att_b[:, None, :],
    )

    def batched(a):
        bs = (1,) + a.shape[1:]
        return pl.BlockSpec(bs, lambda b: (b,) + (0,) * (a.ndim - 1))

    def full(a):
        return pl.BlockSpec(a.shape, lambda b: (0,) * a.ndim)

    in_specs = [batched(o) for o in operands[:4]] + \
               [full(o) for o in operands[4:]]

    out = pl.pallas_call(
        _fwd_kernel,
        grid=(G,),
        in_specs=in_specs,
        out_specs=pl.BlockSpec((1, 2, _N, 3), lambda b: (b, 0, 0, 0)),
        out_shape=jax.ShapeDtypeStruct((G, 2, _N, 3), jnp.float32),
        compiler_params=pltpu.CompilerParams(
            dimension_semantics=("parallel",)),
    )(*operands)
    return out.reshape(B, _N * 3)


# speedup vs baseline: 2.8849x; 1.3878x over previous
import jax
import jax.numpy as jnp
from jax import lax
from jax.experimental import pallas as pl
from jax.experimental.pallas import tpu as pltpu

_N = 55
_P = 56
_H = 64
_H2 = 128
_L = 5
_PP = _P * _P
_CR = 3.0


def _fwd_kernel(t_ref, d_ref, x_ref, xt_ref,
                scisrj_ref, srj_ref, scisci_ref, selT_ref, agg_ref,
                e2b_ref, e56_ref,
                embW_ref, embb_ref,
                eW1a_ref, eW1b_ref, ewrLR_ref, eweLR_ref,
                eb1_ref, eW2_ref, eb2_ref,
                nW1a_ref, nW1b_ref, nb1_ref, nW2_ref, nb2_ref,
                cW1_ref, cb1_ref, cW2_ref,
                aW_ref, ab_ref,
                out_ref):
    f32 = jnp.float32
    silu = jax.nn.silu

    def mm(a, b):
        return jnp.dot(a, b, preferred_element_type=f32)

    jsub = lax.broadcasted_iota(jnp.int32, (_P, _P), 0)
    maskt_cr = (jsub < _N).astype(f32) * _CR

    zc = jnp.zeros((_P - _N, 3), f32)
    zr = jnp.zeros((3, _P - _N), f32)
    x0c = [jnp.concatenate([x_ref[0, s], zc], axis=0) for s in (0, 1)]
    x0r = [jnp.concatenate([xt_ref[0, s], zr], axis=1) for s in (0, 1)]

    h0 = (t_ref[0] * embW_ref[0:1, :] + d_ref[0] * embW_ref[1:2, :]
          + embb_ref[...])
    h = jnp.broadcast_to(
        jnp.concatenate([h0[0:1, :], h0[1:2, :]], axis=1), (_P, _H2))

    def diffs(cc, cr):
        return [cr[k:k + 1, :] - cc[:, k:k + 1] for k in range(3)]

    def radial(e):
        return e[0] * e[0] + e[1] * e[1] + e[2] * e[2]

    e0 = [diffs(x0c[s], x0r[s]) for s in (0, 1)]
    radT0 = [radial(e0[s]) for s in (0, 1)]
    a_ea = mm(srj_ref[...],
              jnp.concatenate(radT0, axis=1)) * scisci_ref[...]

    coord_c = list(x0c)
    coord_r = list(x0r)
    e2, radT = e0, radT0
    for l in range(_L):
        if l:
            e2 = [diffs(coord_c[s], coord_r[s]) for s in (0, 1)]
            radT = [radial(e2[s]) for s in (0, 1)]
        invT = [1.0 / (jnp.sqrt(radT[s]) + 1.0) for s in (0, 1)]

        preI = mm(h, eW1a_ref[l]) + eb1_ref[l]
        preJ = mm(h, eW1b_ref[l])
        a_rad = mm(srj_ref[...],
                   jnp.concatenate(radT, axis=1)) * scisci_ref[...]
        m1 = silu(mm(scisrj_ref[...],
                     jnp.concatenate([preI, preJ], axis=0))
                  + mm(a_rad, ewrLR_ref[l]) + mm(a_ea, eweLR_ref[l]))
        m2 = silu(mm(m1, eW2_ref[l]) + eb2_ref[l])
        att = jax.nn.sigmoid(mm(m2, aW_ref[l]) + ab_ref[l])
        m = m2 * mm(att, e2b_ref[...])
        cm = silu(mm(m, cW1_ref[l]) + cb1_ref[l])
        cms = mm(cm, cW2_ref[l])
        gates = jnp.tanh(
            mm(selT_ref[...],
               mm(cms, e56_ref[...]) * scisci_ref[...]))
        agg = mm(agg_ref[...], m)
        for s in (0, 1):
            ts = gates[:, s * _P:(s + 1) * _P] * invT[s] * maskt_cr
            drow = jnp.concatenate(
                [jnp.sum(e2[s][k] * ts, axis=0, keepdims=True)
                 for k in range(3)], axis=0)
            coord_r[s] = coord_r[s] + drow
            coord_c[s] = coord_c[s] + drow.T

        hn = silu(mm(h, nW1a_ref[l]) + mm(agg, nW1b_ref[l])
                  + nb1_ref[l])
        h = h + mm(hn, nW2_ref[l]) + nb2_ref[l]

    for s in (0, 1):
        vel = (coord_c[s] - x0c[s])[:_N, :]
        vel = vel - jnp.sum(vel, axis=0, keepdims=True) * (1.0 / _N)
        out_ref[0, s] = vel


def _blockdiag(w):
    z = jnp.zeros_like(w)
    return jnp.concatenate(
        [jnp.concatenate([w, z], axis=2),
         jnp.concatenate([z, w], axis=2)], axis=1)


def _lanes2(b):
    return jnp.concatenate([b, b], axis=2)


def kernel(t, x, d_base, emb_W, emb_b, edge_W1, edge_b1, edge_W2, edge_b2,
           node_W1, node_b1, node_W2, node_b2, coord_W1, coord_b1, coord_W2,
           att_W, att_b):
    B = t.shape[0]
    G = B // 2
    x4 = x.reshape(G, 2, _N, 3)
    xt = jnp.swapaxes(x4, 2, 3)
    t3 = t.reshape(G, 2, 1)
    db3 = d_base.reshape(G, 2, 1)

    pidx = jnp.arange(_PP, dtype=jnp.int32)
    pi, pj = pidx // _P, pidx % _P
    k56 = jnp.arange(_P, dtype=jnp.int32)
    sci = (pi[:, None] == k56[None, :]).astype(jnp.float32)
    srj = (pj[:, None] == k56[None, :]).astype(jnp.float32)
    scisrj = jnp.concatenate([sci, srj], axis=1)
    scisci = jnp.concatenate([sci, sci], axis=1)
    selT = srj.T
    agg_sel = ((pi[None, :] == k56[:, None])
               & (pj[None, :] != k56[:, None])
               & (pj[None, :] < _N)).astype(jnp.float32)
    lane = jnp.arange(_H2)
    e2b = jnp.stack([(lane < _H).astype(jnp.float32),
                     (lane >= _H).astype(jnp.float32)], axis=0)
    lane2 = jnp.arange(2 * _P)
    e56 = jnp.stack([(lane2 < _P).astype(jnp.float32),
                     (lane2 >= _P).astype(jnp.float32)], axis=0)

    def half_stack(w):
        z = jnp.zeros_like(w)
        top = jnp.broadcast_to(
            jnp.concatenate([w, z], axis=2), (_L, _P, _H2))
        bot = jnp.broadcast_to(
            jnp.concatenate([z, w], axis=2), (_L, _P, _H2))
        return jnp.concatenate([top, bot], axis=1)

    ewr = edge_W1[:, 2 * _H:2 * _H + 1, :]
    ewe = edge_W1[:, 2 * _H + 1:, :]
    operands = (
        t3, db3, x4, xt,
        scisrj, srj, scisci, selT, agg_sel, e2b, e56,
        emb_W, emb_b.reshape(1, _H),
        _blockdiag(edge_W1[:, :_H, :]), _blockdiag(edge_W1[:, _H:2 * _H, :]),
        half_stack(ewr), half_stack(ewe),
        _lanes2(edge_b1[:, None, :]),
        _blockdiag(edge_W2), _lanes2(edge_b2[:, None, :]),
        _blockdiag(node_W1[:, :_H, :]), _blockdiag(node_W1[:, _H:, :]),
        _lanes2(node_b1[:, None, :]), _blockdiag(node_W2),
        _lanes2(node_b2[:, None, :]),
        _blockdiag(coord_W1), _lanes2(coord_b1[:, None, :]),
        _blockdiag(coord_W2),
        _blockdiag(att_W), att_b[:, None, :],
    )

    def batched(a):
        bs = (1,) + a.shape[1:]
        return pl.BlockSpec(bs, lambda b: (b,) + (0,) * (a.ndim - 1))

    def full(a):
        return pl.BlockSpec(a.shape, lambda b: (0,) * a.ndim)

    in_specs = [batched(o) for o in operands[:4]] + \
               [full(o) for o in operands[4:]]

    out = pl.pallas_call(
        _fwd_kernel,
        grid=(G,),
        in_specs=in_specs,
        out_specs=pl.BlockSpec((1, 2, _N, 3), lambda b: (b, 0, 0, 0)),
        out_shape=jax.ShapeDtypeStruct((G, 2, _N, 3), jnp.float32),
        compiler_params=pltpu.CompilerParams(
            dimension_semantics=("parallel",)),
    )(*operands)
    return out.reshape(B, _N * 3)
